# baseline (speedup 1.0000x reference)
import jax
import jax.numpy as jnp
from jax import lax
from jax.experimental import pallas as pl
from jax.experimental.pallas import tpu as pltpu
from jax.experimental.pallas import tpu_sc as plsc

N = 10000
NP = 10240
E = 320000
NS = 16
CHUNK = 80
EDGES_PER_TILE = E // NS
NCHUNK = EDGES_PER_TILE // CHUNK
NBUF = 5
AHEAD = 3
DW = 8
ROWS_PER_TILE = NP // NS
CP = 64
NCP = ROWS_PER_TILE // CP


def _ring(hn, src_v, dst_v, rows_v, acc_sh, sem_g, sem_s, nchunk, deg=None):
    for b in range(AHEAD):
        pltpu.async_copy(hn.at[src_v.at[b]], rows_v.at[b], sem_g[b])

    def step(g, carry):
        for b in range(NBUF):
            j = g * NBUF + b
            bn = (b + AHEAD) % NBUF
            jn = j + AHEAD

            @pl.when(jn < nchunk)
            def _():
                @pl.when(jn >= NBUF)
                def _():
                    pltpu.make_async_copy(
                        rows_v.at[bn], acc_sh.at[dst_v.at[jn - NBUF]],
                        sem_s[bn]).wait()
                    if deg is not None:
                        ones_v, deg_sh, sem_d = deg
                        pltpu.make_async_copy(
                            ones_v, deg_sh.at[dst_v.at[jn - NBUF]],
                            sem_d[bn]).wait()
                pltpu.async_copy(hn.at[src_v.at[jn]], rows_v.at[bn],
                                 sem_g[bn])

            pltpu.make_async_copy(
                hn.at[src_v.at[j]], rows_v.at[b], sem_g[b]).wait()
            pltpu.async_copy(rows_v.at[b], acc_sh.at[dst_v.at[j]],
                             sem_s[b], add=True)
            if deg is not None:
                ones_v, deg_sh, sem_d = deg
                pltpu.async_copy(ones_v, deg_sh.at[dst_v.at[j]],
                                 sem_d[b], add=True)
        return carry

    lax.fori_loop(0, nchunk // NBUF, step, 0)

    for c in range(nchunk - NBUF, nchunk):
        b = c % NBUF
        pltpu.make_async_copy(
            rows_v.at[b], acc_sh.at[dst_v.at[c]], sem_s[b]).wait()
        if deg is not None:
            ones_v, deg_sh, sem_d = deg
            pltpu.make_async_copy(
                ones_v, deg_sh.at[dst_v.at[c]], sem_d[b]).wait()


def _make_sc_agg(d_half, with_deg):
    mesh = plsc.VectorSubcoreMesh(core_axis_name="c", subcore_axis_name="s")
    out_type = [jax.ShapeDtypeStruct((NP, d_half), jnp.float32),
                jax.ShapeDtypeStruct((NP, d_half), jnp.float32)]
    if with_deg:
        out_type.append(jax.ShapeDtypeStruct((NP, DW), jnp.float32))
    scratch = [
        pltpu.VMEM_SHARED((NP, d_half), jnp.float32),
        pltpu.VMEM((NCHUNK, CHUNK), jnp.int32),
        pltpu.VMEM((NCHUNK, CHUNK), jnp.int32),
        pltpu.VMEM((NBUF, CHUNK, d_half), jnp.float32),
        pltpu.VMEM((CP, d_half), jnp.float32),
    ] + [pltpu.SemaphoreType.DMA] * (2 * NBUF)
    if with_deg:
        scratch += [
            pltpu.VMEM_SHARED((NP, DW), jnp.float32),
            pltpu.VMEM((CHUNK, DW), jnp.float32),
            pltpu.VMEM((CP, DW), jnp.float32),
        ] + [pltpu.SemaphoreType.DMA] * NBUF

    def body(hn_a, hn_b, src_r, dst_r, zrows, *rest):
        if with_deg:
            ones_c = rest[0]
            rest = rest[1:]
        agg_a, agg_b = rest[:2]
        rest = rest[2:]
        if with_deg:
            deg_out = rest[0]
            rest = rest[1:]
        acc_sh, src_v, dst_v, rows_v, zbuf = rest[:5]
        sem_g = rest[5:5 + NBUF]
        sem_s = rest[5 + NBUF:5 + 2 * NBUF]
        if with_deg:
            deg_sh, ones_v, dbuf = rest[5 + 2 * NBUF:5 + 2 * NBUF + 3]
            sem_d = rest[5 + 2 * NBUF + 3:]
        cid = lax.axis_index("c")
        sid = lax.axis_index("s")

        pltpu.sync_copy(src_r.at[sid], src_v)
        pltpu.sync_copy(dst_r.at[sid], dst_v)

        pltpu.sync_copy(zrows, zbuf)
        for q in range(NCP):
            pltpu.sync_copy(zbuf, acc_sh.at[pl.ds(sid * ROWS_PER_TILE + q * CP, CP)])

        if with_deg:
            pltpu.sync_copy(ones_c, ones_v)
            pltpu.sync_copy(zrows.at[:, pl.ds(0, DW)], dbuf)

            @pl.when(cid == 0)
            def _():
                for q in range(NCP):
                    pltpu.sync_copy(
                        dbuf, deg_sh.at[pl.ds(sid * ROWS_PER_TILE + q * CP, CP)])

        plsc.subcore_barrier()

        @pl.when(cid == 0)
        def _():
            _ring(hn_a, src_v, dst_v, rows_v, acc_sh, sem_g, sem_s, NCHUNK,
                  (ones_v, deg_sh, sem_d) if with_deg else None)

        @pl.when(cid == 1)
        def _():
            _ring(hn_b, src_v, dst_v, rows_v, acc_sh, sem_g, sem_s, NCHUNK)

        plsc.subcore_barrier()

        def copyout(agg):
            for q in range(NCP):
                sl = pl.ds(sid * ROWS_PER_TILE + q * CP, CP)
                pltpu.sync_copy(acc_sh.at[sl], agg.at[sl])

        @pl.when(cid == 0)
        def _():
            copyout(agg_a)
            if with_deg:
                for q in range(NCP):
                    sl = pl.ds(sid * ROWS_PER_TILE + q * CP, CP)
                    pltpu.sync_copy(deg_sh.at[sl], deg_out.at[sl])

        @pl.when(cid == 1)
        def _():
            copyout(agg_b)

    return pl.kernel(
        body, out_type=tuple(out_type), mesh=mesh,
        scratch_types=tuple(scratch),
        compiler_params=pltpu.CompilerParams(use_tc_tiling_on_sc=False))


NCHUNK_ES = E // 2 // NS // CHUNK


def _make_sc_es(d):
    mesh = plsc.VectorSubcoreMesh(core_axis_name="c", subcore_axis_name="s")
    out_type = (jax.ShapeDtypeStruct((NP, d), jnp.float32),
                jax.ShapeDtypeStruct((NP, d), jnp.float32))
    scratch = [
        pltpu.VMEM_SHARED((NP, d), jnp.float32),
        pltpu.VMEM((NCHUNK_ES, CHUNK), jnp.int32),
        pltpu.VMEM((NCHUNK_ES, CHUNK), jnp.int32),
        pltpu.VMEM((NBUF, CHUNK, d), jnp.float32),
        pltpu.VMEM((CP, d), jnp.float32),
    ] + [pltpu.SemaphoreType.DMA] * (2 * NBUF)

    def body(hn, src_r, dst_r, zrows, part0, part1,
             acc_sh, src_v, dst_v, rows_v, zbuf, *sems):
        sem_g = sems[:NBUF]
        sem_s = sems[NBUF:]
        cid = lax.axis_index("c")
        sid = lax.axis_index("s")
        wid = cid * NS + sid

        pltpu.sync_copy(src_r.at[wid], src_v)
        pltpu.sync_copy(dst_r.at[wid], dst_v)
        pltpu.sync_copy(zrows, zbuf)
        for q in range(NCP):
            pltpu.sync_copy(
                zbuf, acc_sh.at[pl.ds(sid * ROWS_PER_TILE + q * CP, CP)])
        plsc.subcore_barrier()

        _ring(hn, src_v, dst_v, rows_v, acc_sh, sem_g, sem_s, NCHUNK_ES)

        plsc.subcore_barrier()

        def copyout(part):
            for q in range(NCP):
                sl = pl.ds(sid * ROWS_PER_TILE + q * CP, CP)
                pltpu.sync_copy(acc_sh.at[sl], part.at[sl])

        @pl.when(cid == 0)
        def _():
            copyout(part0)

        @pl.when(cid == 1)
        def _():
            copyout(part1)

    return pl.kernel(
        body, out_type=out_type, mesh=mesh, scratch_types=tuple(scratch),
        compiler_params=pltpu.CompilerParams(use_tc_tiling_on_sc=False))


_sc64_deg = _make_sc_agg(64, True)
_sc64 = _make_sc_agg(64, False)
_sc_es64 = _make_sc_es(64)

_R = 1000


def _tc_first(x, ws, wn, b):
    d_out = ws.shape[1]
    dh = d_out // 2

    def body(x_ref, ws_ref, wn_ref, b_ref, hs_ref, hna_ref, hnb_ref):
        h = x_ref[...]
        hs_ref[...] = jnp.dot(h, ws_ref[...],
                              preferred_element_type=jnp.float32) + b_ref[...]
        hn = jnp.dot(h, wn_ref[...], preferred_element_type=jnp.float32)
        hna_ref[...] = hn[:, :dh]
        hnb_ref[...] = hn[:, dh:]

    d_in = x.shape[1]
    return pl.pallas_call(
        body,
        grid=(N // _R,),
        in_specs=[
            pl.BlockSpec((_R, d_in), lambda i: (i, 0)),
            pl.BlockSpec((d_in, d_out), lambda i: (0, 0)),
            pl.BlockSpec((d_in, d_out), lambda i: (0, 0)),
            pl.BlockSpec((1, d_out), lambda i: (0, 0)),
        ],
        out_specs=[
            pl.BlockSpec((_R, d_out), lambda i: (i, 0)),
            pl.BlockSpec((_R, dh), lambda i: (i, 0)),
            pl.BlockSpec((_R, dh), lambda i: (i, 0)),
        ],
        out_shape=[
            jax.ShapeDtypeStruct((N, d_out), jnp.float32),
            jax.ShapeDtypeStruct((N, dh), jnp.float32),
            jax.ShapeDtypeStruct((N, dh), jnp.float32),
        ],
    )(x, ws, wn, b)


def _tc_mid(hs, aa, ab, deg, ws, wn, b, split=True):
    d_in = hs.shape[1]
    ah = aa.shape[1]
    d_out = ws.shape[1]
    dh = d_out // 2

    def body(hs_ref, aa_ref, ab_ref, deg_ref, ws_ref, wn_ref, b_ref,
             hsn_ref, *hn_refs):
        agg = jnp.concatenate([aa_ref[...], ab_ref[...]], axis=1)
        inv = 1.0 / jnp.maximum(deg_ref[...][:, 0:1], 1.0)
        h = jnp.maximum(hs_ref[...] + agg * inv, 0.0)
        hsn_ref[...] = jnp.dot(h, ws_ref[...],
                               preferred_element_type=jnp.float32) + b_ref[...]
        hn = jnp.dot(h, wn_ref[...], preferred_element_type=jnp.float32)
        if split:
            hn_refs[0][...] = hn[:, :dh]
            hn_refs[1][...] = hn[:, dh:]
        else:
            hn_refs[0][...] = hn

    if split:
        hn_specs = [pl.BlockSpec((_R, dh), lambda i: (i, 0)),
                    pl.BlockSpec((_R, dh), lambda i: (i, 0))]
        hn_shapes = [jax.ShapeDtypeStruct((N, dh), jnp.float32),
                     jax.ShapeDtypeStruct((N, dh), jnp.float32)]
    else:
        hn_specs = [pl.BlockSpec((_R, d_out), lambda i: (i, 0))]
        hn_shapes = [jax.ShapeDtypeStruct((N, d_out), jnp.float32)]

    return pl.pallas_call(
        body,
        grid=(N // _R,),
        in_specs=[
            pl.BlockSpec((_R, d_in), lambda i: (i, 0)),
            pl.BlockSpec((_R, ah), lambda i: (i, 0)),
            pl.BlockSpec((_R, ah), lambda i: (i, 0)),
            pl.BlockSpec((_R, DW), lambda i: (i, 0)),
            pl.BlockSpec((d_in, d_out), lambda i: (0, 0)),
            pl.BlockSpec((d_in, d_out), lambda i: (0, 0)),
            pl.BlockSpec((1, d_out), lambda i: (0, 0)),
        ],
        out_specs=[pl.BlockSpec((_R, d_out), lambda i: (i, 0))] + hn_specs,
        out_shape=[jax.ShapeDtypeStruct((N, d_out), jnp.float32)] + hn_shapes,
    )(hs, aa, ab, deg, ws, wn, b)


def _tc_final(hs, aa, ab, deg):
    d_out = hs.shape[1]
    ah = aa.shape[1]

    def body(hs_ref, aa_ref, ab_ref, deg_ref, o_ref):
        agg = aa_ref[...] + ab_ref[...]
        inv = 1.0 / jnp.maximum(deg_ref[...][:, 0:1], 1.0)
        o_ref[...] = hs_ref[...] + agg * inv

    return pl.pallas_call(
        body,
        grid=(N // _R,),
        in_specs=[
            pl.BlockSpec((_R, d_out), lambda i: (i, 0)),
            pl.BlockSpec((_R, ah), lambda i: (i, 0)),
            pl.BlockSpec((_R, ah), lambda i: (i, 0)),
            pl.BlockSpec((_R, DW), lambda i: (i, 0)),
        ],
        out_specs=pl.BlockSpec((_R, d_out), lambda i: (i, 0)),
        out_shape=jax.ShapeDtypeStruct((N, d_out), jnp.float32),
    )(hs, aa, ab, deg)


def kernel(x, edge_index, W_self0, W_neigh0, b0, W_self1, W_neigh1, b1,
           W_self2, W_neigh2, b2):
    src_r = edge_index[0].reshape(NS, NCHUNK, CHUNK)
    dst_r = edge_index[1].reshape(NS, NCHUNK, CHUNK)
    src_es = edge_index[0].reshape(2 * NS, NCHUNK_ES, CHUNK)
    dst_es = edge_index[1].reshape(2 * NS, NCHUNK_ES, CHUNK)
    z64 = jnp.zeros((CP, 64), jnp.float32)
    ones_c = jnp.ones((CHUNK, DW), jnp.float32)
    hs0, hn0a, hn0b = _tc_first(x, W_self0, W_neigh0, b0.reshape(1, -1))
    agg0a, agg0b, deg = _sc64_deg(hn0a, hn0b, src_r, dst_r, z64, ones_c)
    hs1, hn1a, hn1b = _tc_mid(hs0, agg0a, agg0b, deg,
                              W_self1, W_neigh1, b1.reshape(1, -1))
    agg1a, agg1b = _sc64(hn1a, hn1b, src_r, dst_r, z64)
    hs2, hn2 = _tc_mid(hs1, agg1a, agg1b, deg,
                       W_self2, W_neigh2, b2.reshape(1, -1), split=False)
    p0, p1 = _sc_es64(hn2, src_es, dst_es, z64)
    return _tc_final(hs2, p0, p1, deg)


# speedup vs baseline: 1.0139x; 1.0139x over previous
import jax
import jax.numpy as jnp
from jax import lax
from jax.experimental import pallas as pl
from jax.experimental.pallas import tpu as pltpu
from jax.experimental.pallas import tpu_sc as plsc

N = 10000
NP = 10240
E = 320000
NS = 16
CHUNK = 80
EDGES_PER_TILE = E // NS
NCHUNK = EDGES_PER_TILE // CHUNK
NBUF = 5
AHEAD = 3
DW = 8
ROWS_PER_TILE = NP // NS
CP = 64
NCP = ROWS_PER_TILE // CP


def _ring(hn, src_v, dst_v, rows_v, acc_sh, sem_g, sem_s, nchunk, deg=None):
    for b in range(AHEAD):
        pltpu.async_copy(hn.at[src_v.at[b]], rows_v.at[b], sem_g[b])

    def step(g, carry):
        for b in range(NBUF):
            j = g * NBUF + b
            bn = (b + AHEAD) % NBUF
            jn = j + AHEAD

            @pl.when(jn < nchunk)
            def _():
                @pl.when(jn >= NBUF)
                def _():
                    pltpu.make_async_copy(
                        rows_v.at[bn], acc_sh.at[dst_v.at[jn - NBUF]],
                        sem_s[bn]).wait()
                    if deg is not None:
                        ones_v, deg_sh, sem_d = deg
                        pltpu.make_async_copy(
                            ones_v, deg_sh.at[dst_v.at[jn - NBUF]],
                            sem_d[bn]).wait()
                pltpu.async_copy(hn.at[src_v.at[jn]], rows_v.at[bn],
                                 sem_g[bn])

            pltpu.make_async_copy(
                hn.at[src_v.at[j]], rows_v.at[b], sem_g[b]).wait()
            pltpu.async_copy(rows_v.at[b], acc_sh.at[dst_v.at[j]],
                             sem_s[b], add=True)
            if deg is not None:
                ones_v, deg_sh, sem_d = deg
                pltpu.async_copy(ones_v, deg_sh.at[dst_v.at[j]],
                                 sem_d[b], add=True)
        return carry

    lax.fori_loop(0, nchunk // NBUF, step, 0)

    for c in range(nchunk - NBUF, nchunk):
        b = c % NBUF
        pltpu.make_async_copy(
            rows_v.at[b], acc_sh.at[dst_v.at[c]], sem_s[b]).wait()
        if deg is not None:
            ones_v, deg_sh, sem_d = deg
            pltpu.make_async_copy(
                ones_v, deg_sh.at[dst_v.at[c]], sem_d[b]).wait()


def _make_sc_agg(d_half, with_deg):
    mesh = plsc.VectorSubcoreMesh(core_axis_name="c", subcore_axis_name="s")
    out_type = [jax.ShapeDtypeStruct((NP, d_half), jnp.float32),
                jax.ShapeDtypeStruct((NP, d_half), jnp.float32)]
    if with_deg:
        out_type.append(jax.ShapeDtypeStruct((NP, DW), jnp.float32))
    scratch = [
        pltpu.VMEM_SHARED((NP, d_half), jnp.float32),
        pltpu.VMEM((NCHUNK, CHUNK), jnp.int32),
        pltpu.VMEM((NCHUNK, CHUNK), jnp.int32),
        pltpu.VMEM((NBUF, CHUNK, d_half), jnp.float32),
        pltpu.VMEM((CP, d_half), jnp.float32),
    ] + [pltpu.SemaphoreType.DMA] * (2 * NBUF)
    if with_deg:
        scratch += [
            pltpu.VMEM_SHARED((NP, DW), jnp.float32),
            pltpu.VMEM((CHUNK, DW), jnp.float32),
            pltpu.VMEM((CP, DW), jnp.float32),
        ] + [pltpu.SemaphoreType.DMA] * NBUF

    def body(hn_a, hn_b, src_r, dst_r, zrows, *rest):
        if with_deg:
            ones_c = rest[0]
            rest = rest[1:]
        agg_a, agg_b = rest[:2]
        rest = rest[2:]
        if with_deg:
            deg_out = rest[0]
            rest = rest[1:]
        acc_sh, src_v, dst_v, rows_v, zbuf = rest[:5]
        sem_g = rest[5:5 + NBUF]
        sem_s = rest[5 + NBUF:5 + 2 * NBUF]
        if with_deg:
            deg_sh, ones_v, dbuf = rest[5 + 2 * NBUF:5 + 2 * NBUF + 3]
            sem_d = rest[5 + 2 * NBUF + 3:]
        cid = lax.axis_index("c")
        sid = lax.axis_index("s")

        pltpu.sync_copy(src_r.at[sid], src_v)
        pltpu.sync_copy(dst_r.at[sid], dst_v)

        pltpu.sync_copy(zrows, zbuf)
        for q in range(NCP):
            pltpu.sync_copy(zbuf, acc_sh.at[pl.ds(sid * ROWS_PER_TILE + q * CP, CP)])

        if with_deg:
            pltpu.sync_copy(ones_c, ones_v)
            pltpu.sync_copy(zrows.at[:, pl.ds(0, DW)], dbuf)

            @pl.when(cid == 0)
            def _():
                for q in range(NCP):
                    pltpu.sync_copy(
                        dbuf, deg_sh.at[pl.ds(sid * ROWS_PER_TILE + q * CP, CP)])

        plsc.subcore_barrier()

        @pl.when(cid == 0)
        def _():
            _ring(hn_a, src_v, dst_v, rows_v, acc_sh, sem_g, sem_s, NCHUNK,
                  (ones_v, deg_sh, sem_d) if with_deg else None)

        @pl.when(cid == 1)
        def _():
            _ring(hn_b, src_v, dst_v, rows_v, acc_sh, sem_g, sem_s, NCHUNK)

        plsc.subcore_barrier()

        def copyout(agg):
            for q in range(NCP):
                sl = pl.ds(sid * ROWS_PER_TILE + q * CP, CP)
                pltpu.sync_copy(acc_sh.at[sl], agg.at[sl])

        @pl.when(cid == 0)
        def _():
            copyout(agg_a)
            if with_deg:
                for q in range(NCP):
                    sl = pl.ds(sid * ROWS_PER_TILE + q * CP, CP)
                    pltpu.sync_copy(deg_sh.at[sl], deg_out.at[sl])

        @pl.when(cid == 1)
        def _():
            copyout(agg_b)

    return pl.kernel(
        body, out_type=tuple(out_type), mesh=mesh,
        scratch_types=tuple(scratch),
        compiler_params=pltpu.CompilerParams(use_tc_tiling_on_sc=False))


NCHUNK_ES = E // 2 // NS // CHUNK


def _make_sc_es(d):
    mesh = plsc.VectorSubcoreMesh(core_axis_name="c", subcore_axis_name="s")
    out_type = (jax.ShapeDtypeStruct((NP, d), jnp.float32),
                jax.ShapeDtypeStruct((NP, d), jnp.float32))
    scratch = [
        pltpu.VMEM_SHARED((NP, d), jnp.float32),
        pltpu.VMEM((NCHUNK_ES, CHUNK), jnp.int32),
        pltpu.VMEM((NCHUNK_ES, CHUNK), jnp.int32),
        pltpu.VMEM((NBUF, CHUNK, d), jnp.float32),
        pltpu.VMEM((CP, d), jnp.float32),
    ] + [pltpu.SemaphoreType.DMA] * (2 * NBUF)

    def body(hn, src_r, dst_r, zrows, part0, part1,
             acc_sh, src_v, dst_v, rows_v, zbuf, *sems):
        sem_g = sems[:NBUF]
        sem_s = sems[NBUF:]
        cid = lax.axis_index("c")
        sid = lax.axis_index("s")
        wid = cid * NS + sid

        pltpu.sync_copy(src_r.at[wid], src_v)
        pltpu.sync_copy(dst_r.at[wid], dst_v)
        pltpu.sync_copy(zrows, zbuf)
        for q in range(NCP):
            pltpu.sync_copy(
                zbuf, acc_sh.at[pl.ds(sid * ROWS_PER_TILE + q * CP, CP)])
        plsc.subcore_barrier()

        _ring(hn, src_v, dst_v, rows_v, acc_sh, sem_g, sem_s, NCHUNK_ES)

        plsc.subcore_barrier()

        def copyout(part):
            for q in range(NCP):
                sl = pl.ds(sid * ROWS_PER_TILE + q * CP, CP)
                pltpu.sync_copy(acc_sh.at[sl], part.at[sl])

        @pl.when(cid == 0)
        def _():
            copyout(part0)

        @pl.when(cid == 1)
        def _():
            copyout(part1)

    return pl.kernel(
        body, out_type=out_type, mesh=mesh, scratch_types=tuple(scratch),
        compiler_params=pltpu.CompilerParams(use_tc_tiling_on_sc=False))


_sc64_deg = _make_sc_agg(64, True)
_sc64 = _make_sc_agg(64, False)
_sc_es64 = _make_sc_es(64)

_R = 1000


def _tc_self(x, ws, b):
    d_in = x.shape[1]
    d_out = ws.shape[1]

    def body(x_ref, ws_ref, b_ref, o_ref):
        o_ref[...] = jnp.dot(x_ref[...], ws_ref[...],
                             preferred_element_type=jnp.float32) + b_ref[...]

    return pl.pallas_call(
        body,
        grid=(N // _R,),
        in_specs=[
            pl.BlockSpec((_R, d_in), lambda i: (i, 0)),
            pl.BlockSpec((d_in, d_out), lambda i: (0, 0)),
            pl.BlockSpec((1, d_out), lambda i: (0, 0)),
        ],
        out_specs=pl.BlockSpec((_R, d_out), lambda i: (i, 0)),
        out_shape=jax.ShapeDtypeStruct((N, d_out), jnp.float32),
    )(x, ws, b)


def _tc_mid2(hs, aa, ab, deg, wnp, ws, wn, b):
    d_in = hs.shape[1]
    ah = aa.shape[1]
    d_out = ws.shape[1]
    dh = d_out // 2

    def body(hs_ref, aa_ref, ab_ref, deg_ref, wnp_ref, ws_ref, wn_ref, b_ref,
             hsn_ref, hna_ref, hnb_ref):
        agg = jnp.concatenate([aa_ref[...], ab_ref[...]], axis=1)
        inv = 1.0 / jnp.maximum(deg_ref[...][:, 0:1], 1.0)
        neigh = jnp.dot(agg * inv, wnp_ref[...],
                        preferred_element_type=jnp.float32)
        h = jnp.maximum(hs_ref[...] + neigh, 0.0)
        hsn_ref[...] = jnp.dot(h, ws_ref[...],
                               preferred_element_type=jnp.float32) + b_ref[...]
        hn = jnp.dot(h, wn_ref[...], preferred_element_type=jnp.float32)
        hna_ref[...] = hn[:, :dh]
        hnb_ref[...] = hn[:, dh:]

    return pl.pallas_call(
        body,
        grid=(N // _R,),
        in_specs=[
            pl.BlockSpec((_R, d_in), lambda i: (i, 0)),
            pl.BlockSpec((_R, ah), lambda i: (i, 0)),
            pl.BlockSpec((_R, ah), lambda i: (i, 0)),
            pl.BlockSpec((_R, DW), lambda i: (i, 0)),
            pl.BlockSpec((2 * ah, d_in), lambda i: (0, 0)),
            pl.BlockSpec((d_in, d_out), lambda i: (0, 0)),
            pl.BlockSpec((d_in, d_out), lambda i: (0, 0)),
            pl.BlockSpec((1, d_out), lambda i: (0, 0)),
        ],
        out_specs=[
            pl.BlockSpec((_R, d_out), lambda i: (i, 0)),
            pl.BlockSpec((_R, dh), lambda i: (i, 0)),
            pl.BlockSpec((_R, dh), lambda i: (i, 0)),
        ],
        out_shape=[
            jax.ShapeDtypeStruct((N, d_out), jnp.float32),
            jax.ShapeDtypeStruct((N, dh), jnp.float32),
            jax.ShapeDtypeStruct((N, dh), jnp.float32),
        ],
    )(hs, aa, ab, deg, wnp, ws, wn, b)


def _tc_mid(hs, aa, ab, deg, ws, wn, b, split=True):
    d_in = hs.shape[1]
    ah = aa.shape[1]
    d_out = ws.shape[1]
    dh = d_out // 2

    def body(hs_ref, aa_ref, ab_ref, deg_ref, ws_ref, wn_ref, b_ref,
             hsn_ref, *hn_refs):
        agg = jnp.concatenate([aa_ref[...], ab_ref[...]], axis=1)
        inv = 1.0 / jnp.maximum(deg_ref[...][:, 0:1], 1.0)
        h = jnp.maximum(hs_ref[...] + agg * inv, 0.0)
        hsn_ref[...] = jnp.dot(h, ws_ref[...],
                               preferred_element_type=jnp.float32) + b_ref[...]
        hn = jnp.dot(h, wn_ref[...], preferred_element_type=jnp.float32)
        if split:
            hn_refs[0][...] = hn[:, :dh]
            hn_refs[1][...] = hn[:, dh:]
        else:
            hn_refs[0][...] = hn

    if split:
        hn_specs = [pl.BlockSpec((_R, dh), lambda i: (i, 0)),
                    pl.BlockSpec((_R, dh), lambda i: (i, 0))]
        hn_shapes = [jax.ShapeDtypeStruct((N, dh), jnp.float32),
                     jax.ShapeDtypeStruct((N, dh), jnp.float32)]
    else:
        hn_specs = [pl.BlockSpec((_R, d_out), lambda i: (i, 0))]
        hn_shapes = [jax.ShapeDtypeStruct((N, d_out), jnp.float32)]

    return pl.pallas_call(
        body,
        grid=(N // _R,),
        in_specs=[
            pl.BlockSpec((_R, d_in), lambda i: (i, 0)),
            pl.BlockSpec((_R, ah), lambda i: (i, 0)),
            pl.BlockSpec((_R, ah), lambda i: (i, 0)),
            pl.BlockSpec((_R, DW), lambda i: (i, 0)),
            pl.BlockSpec((d_in, d_out), lambda i: (0, 0)),
            pl.BlockSpec((d_in, d_out), lambda i: (0, 0)),
            pl.BlockSpec((1, d_out), lambda i: (0, 0)),
        ],
        out_specs=[pl.BlockSpec((_R, d_out), lambda i: (i, 0))] + hn_specs,
        out_shape=[jax.ShapeDtypeStruct((N, d_out), jnp.float32)] + hn_shapes,
    )(hs, aa, ab, deg, ws, wn, b)


def _tc_final(hs, aa, ab, deg):
    d_out = hs.shape[1]
    ah = aa.shape[1]

    def body(hs_ref, aa_ref, ab_ref, deg_ref, o_ref):
        agg = aa_ref[...] + ab_ref[...]
        inv = 1.0 / jnp.maximum(deg_ref[...][:, 0:1], 1.0)
        o_ref[...] = hs_ref[...] + agg * inv

    return pl.pallas_call(
        body,
        grid=(N // _R,),
        in_specs=[
            pl.BlockSpec((_R, d_out), lambda i: (i, 0)),
            pl.BlockSpec((_R, ah), lambda i: (i, 0)),
            pl.BlockSpec((_R, ah), lambda i: (i, 0)),
            pl.BlockSpec((_R, DW), lambda i: (i, 0)),
        ],
        out_specs=pl.BlockSpec((_R, d_out), lambda i: (i, 0)),
        out_shape=jax.ShapeDtypeStruct((N, d_out), jnp.float32),
    )(hs, aa, ab, deg)


def kernel(x, edge_index, W_self0, W_neigh0, b0, W_self1, W_neigh1, b1,
           W_self2, W_neigh2, b2):
    src_r = edge_index[0].reshape(NS, NCHUNK, CHUNK)
    dst_r = edge_index[1].reshape(NS, NCHUNK, CHUNK)
    src_es = edge_index[0].reshape(2 * NS, NCHUNK_ES, CHUNK)
    dst_es = edge_index[1].reshape(2 * NS, NCHUNK_ES, CHUNK)
    z64 = jnp.zeros((CP, 64), jnp.float32)
    ones_c = jnp.ones((CHUNK, DW), jnp.float32)
    aggxa, aggxb, deg = _sc64_deg(x[:, :64], x[:, 64:], src_r, dst_r,
                                  z64, ones_c)
    hs0 = _tc_self(x, W_self0, b0.reshape(1, -1))
    hs1, hn1a, hn1b = _tc_mid2(hs0, aggxa, aggxb, deg, W_neigh0,
                               W_self1, W_neigh1, b1.reshape(1, -1))
    agg1a, agg1b = _sc64(hn1a, hn1b, src_r, dst_r, z64)
    hs2, hn2 = _tc_mid(hs1, agg1a, agg1b, deg,
                       W_self2, W_neigh2, b2.reshape(1, -1), split=False)
    p0, p1 = _sc_es64(hn2, src_es, dst_es, z64)
    return _tc_final(hs2, p0, p1, deg)


# speedup vs baseline: 1.0301x; 1.0159x over previous
import jax
import jax.numpy as jnp
from jax import lax
from jax.experimental import pallas as pl
from jax.experimental.pallas import tpu as pltpu
from jax.experimental.pallas import tpu_sc as plsc

N = 10000
NP = 10240
E = 320000
NS = 16
CHUNK = 80
EDGES_PER_TILE = E // NS
NCHUNK = EDGES_PER_TILE // CHUNK
NBUF = 5
AHEAD = 3
DW = 8
ROWS_PER_TILE = NP // NS
CP = 64
NCP = ROWS_PER_TILE // CP


def _ring(hn, src_v, dst_v, rows_v, acc_sh, sem_g, sem_s, nchunk, deg=None):
    for b in range(AHEAD):
        pltpu.async_copy(hn.at[src_v.at[b]], rows_v.at[b], sem_g[b])

    def step(g, carry):
        for b in range(NBUF):
            j = g * NBUF + b
            bn = (b + AHEAD) % NBUF
            jn = j + AHEAD

            @pl.when(jn < nchunk)
            def _():
                @pl.when(jn >= NBUF)
                def _():
                    pltpu.make_async_copy(
                        rows_v.at[bn], acc_sh.at[dst_v.at[jn - NBUF]],
                        sem_s[bn]).wait()
                    if deg is not None:
                        ones_v, deg_sh, sem_d = deg
                        pltpu.make_async_copy(
                            ones_v, deg_sh.at[dst_v.at[jn - NBUF]],
                            sem_d[bn]).wait()
                pltpu.async_copy(hn.at[src_v.at[jn]], rows_v.at[bn],
                                 sem_g[bn])

            pltpu.make_async_copy(
                hn.at[src_v.at[j]], rows_v.at[b], sem_g[b]).wait()
            pltpu.async_copy(rows_v.at[b], acc_sh.at[dst_v.at[j]],
                             sem_s[b], add=True)
            if deg is not None:
                ones_v, deg_sh, sem_d = deg
                pltpu.async_copy(ones_v, deg_sh.at[dst_v.at[j]],
                                 sem_d[b], add=True)
        return carry

    lax.fori_loop(0, nchunk // NBUF, step, 0)

    for c in range(nchunk - NBUF, nchunk):
        b = c % NBUF
        pltpu.make_async_copy(
            rows_v.at[b], acc_sh.at[dst_v.at[c]], sem_s[b]).wait()
        if deg is not None:
            ones_v, deg_sh, sem_d = deg
            pltpu.make_async_copy(
                ones_v, deg_sh.at[dst_v.at[c]], sem_d[b]).wait()


def _make_sc_agg(d_half, with_deg):
    mesh = plsc.VectorSubcoreMesh(core_axis_name="c", subcore_axis_name="s")
    out_type = [jax.ShapeDtypeStruct((NP, d_half), jnp.float32),
                jax.ShapeDtypeStruct((NP, d_half), jnp.float32)]
    if with_deg:
        out_type.append(jax.ShapeDtypeStruct((NP, DW), jnp.float32))
    scratch = [
        pltpu.VMEM_SHARED((NP, d_half), jnp.float32),
        pltpu.VMEM((NCHUNK, CHUNK), jnp.int32),
        pltpu.VMEM((NCHUNK, CHUNK), jnp.int32),
        pltpu.VMEM((NBUF, CHUNK, d_half), jnp.float32),
        pltpu.VMEM((CP, d_half), jnp.float32),
    ] + [pltpu.SemaphoreType.DMA] * (2 * NBUF)
    if with_deg:
        scratch += [
            pltpu.VMEM_SHARED((NP, DW), jnp.float32),
            pltpu.VMEM((CHUNK, DW), jnp.float32),
            pltpu.VMEM((CP, DW), jnp.float32),
        ] + [pltpu.SemaphoreType.DMA] * NBUF

    def body(hn_a, hn_b, src_r, dst_r, zrows, *rest):
        if with_deg:
            ones_c = rest[0]
            rest = rest[1:]
        agg_a, agg_b = rest[:2]
        rest = rest[2:]
        if with_deg:
            deg_out = rest[0]
            rest = rest[1:]
        acc_sh, src_v, dst_v, rows_v, zbuf = rest[:5]
        sem_g = rest[5:5 + NBUF]
        sem_s = rest[5 + NBUF:5 + 2 * NBUF]
        if with_deg:
            deg_sh, ones_v, dbuf = rest[5 + 2 * NBUF:5 + 2 * NBUF + 3]
            sem_d = rest[5 + 2 * NBUF + 3:]
        cid = lax.axis_index("c")
        sid = lax.axis_index("s")

        pltpu.sync_copy(src_r.at[sid], src_v)
        pltpu.sync_copy(dst_r.at[sid], dst_v)

        pltpu.sync_copy(zrows, zbuf)
        for q in range(NCP):
            pltpu.sync_copy(zbuf, acc_sh.at[pl.ds(sid * ROWS_PER_TILE + q * CP, CP)])

        if with_deg:
            pltpu.sync_copy(ones_c, ones_v)
            pltpu.sync_copy(zrows.at[:, pl.ds(0, DW)], dbuf)

            @pl.when(cid == 0)
            def _():
                for q in range(NCP):
                    pltpu.sync_copy(
                        dbuf, deg_sh.at[pl.ds(sid * ROWS_PER_TILE + q * CP, CP)])

        plsc.subcore_barrier()

        @pl.when(cid == 0)
        def _():
            _ring(hn_a, src_v, dst_v, rows_v, acc_sh, sem_g, sem_s, NCHUNK,
                  (ones_v, deg_sh, sem_d) if with_deg else None)

        @pl.when(cid == 1)
        def _():
            _ring(hn_b, src_v, dst_v, rows_v, acc_sh, sem_g, sem_s, NCHUNK)

        plsc.subcore_barrier()

        def copyout(agg):
            for q in range(NCP):
                sl = pl.ds(sid * ROWS_PER_TILE + q * CP, CP)
                pltpu.sync_copy(acc_sh.at[sl], agg.at[sl])

        @pl.when(cid == 0)
        def _():
            copyout(agg_a)
            if with_deg:
                for q in range(NCP):
                    sl = pl.ds(sid * ROWS_PER_TILE + q * CP, CP)
                    pltpu.sync_copy(deg_sh.at[sl], deg_out.at[sl])

        @pl.when(cid == 1)
        def _():
            copyout(agg_b)

    return pl.kernel(
        body, out_type=tuple(out_type), mesh=mesh,
        scratch_types=tuple(scratch),
        compiler_params=pltpu.CompilerParams(use_tc_tiling_on_sc=False))


NCHUNK_ES = E // 2 // NS // CHUNK


def _make_sc_es(d):
    mesh = plsc.VectorSubcoreMesh(core_axis_name="c", subcore_axis_name="s")
    out_type = (jax.ShapeDtypeStruct((NP, d), jnp.float32),
                jax.ShapeDtypeStruct((NP, d), jnp.float32))
    scratch = [
        pltpu.VMEM_SHARED((NP, d), jnp.float32),
        pltpu.VMEM((NCHUNK_ES, CHUNK), jnp.int32),
        pltpu.VMEM((NCHUNK_ES, CHUNK), jnp.int32),
        pltpu.VMEM((NBUF, CHUNK, d), jnp.float32),
        pltpu.VMEM((CP, d), jnp.float32),
    ] + [pltpu.SemaphoreType.DMA] * (2 * NBUF)

    def body(hn, src_r, dst_r, zrows, part0, part1,
             acc_sh, src_v, dst_v, rows_v, zbuf, *sems):
        sem_g = sems[:NBUF]
        sem_s = sems[NBUF:]
        cid = lax.axis_index("c")
        sid = lax.axis_index("s")
        wid = cid * NS + sid

        pltpu.sync_copy(src_r.at[wid], src_v)
        pltpu.sync_copy(dst_r.at[wid], dst_v)
        pltpu.sync_copy(zrows, zbuf)
        for q in range(NCP):
            pltpu.sync_copy(
                zbuf, acc_sh.at[pl.ds(sid * ROWS_PER_TILE + q * CP, CP)])
        plsc.subcore_barrier()

        _ring(hn, src_v, dst_v, rows_v, acc_sh, sem_g, sem_s, NCHUNK_ES)

        plsc.subcore_barrier()

        def copyout(part):
            for q in range(NCP):
                sl = pl.ds(sid * ROWS_PER_TILE + q * CP, CP)
                pltpu.sync_copy(acc_sh.at[sl], part.at[sl])

        @pl.when(cid == 0)
        def _():
            copyout(part0)

        @pl.when(cid == 1)
        def _():
            copyout(part1)

    return pl.kernel(
        body, out_type=out_type, mesh=mesh, scratch_types=tuple(scratch),
        compiler_params=pltpu.CompilerParams(use_tc_tiling_on_sc=False))


_sc64_deg = _make_sc_agg(64, True)
_sc64 = _make_sc_agg(64, False)
_sc_es64 = _make_sc_es(64)

_R = 2000


def _tc_self(x, ws, b):
    d_in = x.shape[1]
    d_out = ws.shape[1]

    def body(x_ref, ws_ref, b_ref, o_ref):
        o_ref[...] = jnp.dot(x_ref[...], ws_ref[...],
                             preferred_element_type=jnp.float32) + b_ref[...]

    return pl.pallas_call(
        body,
        grid=(N // _R,),
        in_specs=[
            pl.BlockSpec((_R, d_in), lambda i: (i, 0)),
            pl.BlockSpec((d_in, d_out), lambda i: (0, 0)),
            pl.BlockSpec((1, d_out), lambda i: (0, 0)),
        ],
        out_specs=pl.BlockSpec((_R, d_out), lambda i: (i, 0)),
        out_shape=jax.ShapeDtypeStruct((N, d_out), jnp.float32),
    )(x, ws, b)


def _tc_mid2(hs, aa, ab, deg, wnp, ws, wn, b):
    d_in = hs.shape[1]
    ah = aa.shape[1]
    d_out = ws.shape[1]
    dh = d_out // 2

    def body(hs_ref, aa_ref, ab_ref, deg_ref, wnp_ref, ws_ref, wn_ref, b_ref,
             hsn_ref, hna_ref, hnb_ref):
        agg = jnp.concatenate([aa_ref[...], ab_ref[...]], axis=1)
        inv = 1.0 / jnp.maximum(deg_ref[...][:, 0:1], 1.0)
        neigh = jnp.dot(agg * inv, wnp_ref[...],
                        preferred_element_type=jnp.float32)
        h = jnp.maximum(hs_ref[...] + neigh, 0.0)
        hsn_ref[...] = jnp.dot(h, ws_ref[...],
                               preferred_element_type=jnp.float32) + b_ref[...]
        hn = jnp.dot(h, wn_ref[...], preferred_element_type=jnp.float32)
        hna_ref[...] = hn[:, :dh]
        hnb_ref[...] = hn[:, dh:]

    return pl.pallas_call(
        body,
        grid=(N // _R,),
        in_specs=[
            pl.BlockSpec((_R, d_in), lambda i: (i, 0)),
            pl.BlockSpec((_R, ah), lambda i: (i, 0)),
            pl.BlockSpec((_R, ah), lambda i: (i, 0)),
            pl.BlockSpec((_R, DW), lambda i: (i, 0)),
            pl.BlockSpec((2 * ah, d_in), lambda i: (0, 0)),
            pl.BlockSpec((d_in, d_out), lambda i: (0, 0)),
            pl.BlockSpec((d_in, d_out), lambda i: (0, 0)),
            pl.BlockSpec((1, d_out), lambda i: (0, 0)),
        ],
        out_specs=[
            pl.BlockSpec((_R, d_out), lambda i: (i, 0)),
            pl.BlockSpec((_R, dh), lambda i: (i, 0)),
            pl.BlockSpec((_R, dh), lambda i: (i, 0)),
        ],
        out_shape=[
            jax.ShapeDtypeStruct((N, d_out), jnp.float32),
            jax.ShapeDtypeStruct((N, dh), jnp.float32),
            jax.ShapeDtypeStruct((N, dh), jnp.float32),
        ],
    )(hs, aa, ab, deg, wnp, ws, wn, b)


def _tc_mid(hs, aa, ab, deg, ws, wn, b, split=True):
    d_in = hs.shape[1]
    ah = aa.shape[1]
    d_out = ws.shape[1]
    dh = d_out // 2

    def body(hs_ref, aa_ref, ab_ref, deg_ref, ws_ref, wn_ref, b_ref,
             hsn_ref, *hn_refs):
        agg = jnp.concatenate([aa_ref[...], ab_ref[...]], axis=1)
        inv = 1.0 / jnp.maximum(deg_ref[...][:, 0:1], 1.0)
        h = jnp.maximum(hs_ref[...] + agg * inv, 0.0)
        hsn_ref[...] = jnp.dot(h, ws_ref[...],
                               preferred_element_type=jnp.float32) + b_ref[...]
        hn = jnp.dot(h, wn_ref[...], preferred_element_type=jnp.float32)
        if split:
            hn_refs[0][...] = hn[:, :dh]
            hn_refs[1][...] = hn[:, dh:]
        else:
            hn_refs[0][...] = hn

    if split:
        hn_specs = [pl.BlockSpec((_R, dh), lambda i: (i, 0)),
                    pl.BlockSpec((_R, dh), lambda i: (i, 0))]
        hn_shapes = [jax.ShapeDtypeStruct((N, dh), jnp.float32),
                     jax.ShapeDtypeStruct((N, dh), jnp.float32)]
    else:
        hn_specs = [pl.BlockSpec((_R, d_out), lambda i: (i, 0))]
        hn_shapes = [jax.ShapeDtypeStruct((N, d_out), jnp.float32)]

    return pl.pallas_call(
        body,
        grid=(N // _R,),
        in_specs=[
            pl.BlockSpec((_R, d_in), lambda i: (i, 0)),
            pl.BlockSpec((_R, ah), lambda i: (i, 0)),
            pl.BlockSpec((_R, ah), lambda i: (i, 0)),
            pl.BlockSpec((_R, DW), lambda i: (i, 0)),
            pl.BlockSpec((d_in, d_out), lambda i: (0, 0)),
            pl.BlockSpec((d_in, d_out), lambda i: (0, 0)),
            pl.BlockSpec((1, d_out), lambda i: (0, 0)),
        ],
        out_specs=[pl.BlockSpec((_R, d_out), lambda i: (i, 0))] + hn_specs,
        out_shape=[jax.ShapeDtypeStruct((N, d_out), jnp.float32)] + hn_shapes,
    )(hs, aa, ab, deg, ws, wn, b)


def _tc_final(hs, aa, ab, deg):
    d_out = hs.shape[1]
    ah = aa.shape[1]

    def body(hs_ref, aa_ref, ab_ref, deg_ref, o_ref):
        agg = aa_ref[...] + ab_ref[...]
        inv = 1.0 / jnp.maximum(deg_ref[...][:, 0:1], 1.0)
        o_ref[...] = hs_ref[...] + agg * inv

    return pl.pallas_call(
        body,
        grid=(N // _R,),
        in_specs=[
            pl.BlockSpec((_R, d_out), lambda i: (i, 0)),
            pl.BlockSpec((_R, ah), lambda i: (i, 0)),
            pl.BlockSpec((_R, ah), lambda i: (i, 0)),
            pl.BlockSpec((_R, DW), lambda i: (i, 0)),
        ],
        out_specs=pl.BlockSpec((_R, d_out), lambda i: (i, 0)),
        out_shape=jax.ShapeDtypeStruct((N, d_out), jnp.float32),
    )(hs, aa, ab, deg)


def kernel(x, edge_index, W_self0, W_neigh0, b0, W_self1, W_neigh1, b1,
           W_self2, W_neigh2, b2):
    src_r = edge_index[0].reshape(NS, NCHUNK, CHUNK)
    dst_r = edge_index[1].reshape(NS, NCHUNK, CHUNK)
    src_es = edge_index[0].reshape(2 * NS, NCHUNK_ES, CHUNK)
    dst_es = edge_index[1].reshape(2 * NS, NCHUNK_ES, CHUNK)
    z64 = jnp.zeros((CP, 64), jnp.float32)
    ones_c = jnp.ones((CHUNK, DW), jnp.float32)
    aggxa, aggxb, deg = _sc64_deg(x[:, :64], x[:, 64:], src_r, dst_r,
                                  z64, ones_c)
    hs0 = _tc_self(x, W_self0, b0.reshape(1, -1))
    hs1, hn1a, hn1b = _tc_mid2(hs0, aggxa, aggxb, deg, W_neigh0,
                               W_self1, W_neigh1, b1.reshape(1, -1))
    agg1a, agg1b = _sc64(hn1a, hn1b, src_r, dst_r, z64)
    hs2, hn2 = _tc_mid(hs1, agg1a, agg1b, deg,
                       W_self2, W_neigh2, b2.reshape(1, -1), split=False)
    p0, p1 = _sc_es64(hn2, src_es, dst_es, z64)
    return _tc_final(hs2, p0, p1, deg)
